# Initial kernel scaffold; baseline (speedup 1.0000x reference)
import jax
import jax.numpy as jnp
from jax.experimental import pallas as pl


def kernel(batch_rel_pos, batch_time_pos, rel_emb, time_emb, W1, b1, W2, b2):
    raise NotImplementedError("write your pallas kernel here")


# speedup vs baseline: 100.5548x; 100.5548x over previous
import dataclasses
import functools

import jax
import jax.numpy as jnp
from jax import lax
from jax.experimental import pallas as pl
from jax.experimental.pallas import tpu as pltpu
from jax.experimental.pallas import tpu_sc as plsc

EMBED = 64
NUM_REL = 11
NUM_DT = 51
TBL_W = 64
NC = 2
NS = 16
LANES = 16
NW = NC * NS

BATCH = 16384
SEQ = 50
TOTAL = BATCH * SEQ
PER_W = TOTAL // NW


def _table_body(rel_emb_ref, time_emb_ref, wa_ref, wb_ref, b1_ref, w2_ref,
                b2_ref, out_ref):
    p1 = lax.dot_general(
        rel_emb_ref[...], wa_ref[...], (((1,), (1,)), ((), ())),
        preferred_element_type=jnp.float32, precision=lax.Precision.HIGHEST)
    p2 = lax.dot_general(
        time_emb_ref[...], wb_ref[...], (((1,), (1,)), ((), ())),
        preferred_element_type=jnp.float32, precision=lax.Precision.HIGHEST)
    hidden = jnp.tanh(p1[:, None, :] + p2[None, :, :] + b1_ref[...])
    pre = jnp.sum(hidden * w2_ref[...], axis=-1) + b2_ref[0, 0]
    table = jnp.tanh(pre)
    r_io = lax.broadcasted_iota(jnp.int32, (NUM_REL, TBL_W), 0)
    d_io = lax.broadcasted_iota(jnp.int32, (NUM_REL, TBL_W), 1)
    valid = (d_io < NUM_DT) & ((r_io != 0) | (d_io != 0))
    out_ref[...] = jnp.where(valid, table, 0.0)


_table_call = pl.pallas_call(
    _table_body,
    out_shape=jax.ShapeDtypeStruct((NUM_REL, TBL_W), jnp.float32),
)


def _gather_body(table_hbm, rel_hbm, dt_hbm, out_hbm,
                 table_v, rel_v, dt_v, out_v, sem):
    wid = lax.axis_index("s") * NC + lax.axis_index("c")
    base = wid * PER_W
    cp_t = pltpu.async_copy(table_hbm, table_v, sem)
    cp_r = pltpu.async_copy(rel_hbm.at[pl.ds(base, PER_W)], rel_v, sem)
    cp_d = pltpu.async_copy(dt_hbm.at[pl.ds(base, PER_W)], dt_v, sem)
    cp_t.wait()
    cp_r.wait()
    cp_d.wait()

    @pl.loop(0, PER_W, step=LANES)
    def _(i):
        r = rel_v[pl.ds(i, LANES)]
        d = dt_v[pl.ds(i, LANES)]
        r = jnp.minimum(jnp.maximum(r, 0), NUM_REL - 1)
        d = jnp.minimum(jnp.maximum(d, 0), NUM_DT - 1)
        idx = r * TBL_W + d
        out_v[pl.ds(i, LANES)] = plsc.load_gather(table_v, [idx])

    pltpu.sync_copy(out_v, out_hbm.at[pl.ds(base, PER_W)])


@functools.cache
def _make_gather_kernel():
    cp = pltpu.CompilerParams()
    if "needs_layout_passes" in pltpu.CompilerParams.__dataclass_fields__:
        cp = dataclasses.replace(cp, needs_layout_passes=False)
    return pl.kernel(
        _gather_body,
        compiler_params=cp,
        out_type=jax.ShapeDtypeStruct((TOTAL,), jnp.float32),
        mesh=plsc.VectorSubcoreMesh(core_axis_name="c", subcore_axis_name="s",
                                    num_cores=NC, num_subcores=NS),
        scratch_types=[
            pltpu.VMEM((NUM_REL * TBL_W,), jnp.float32),
            pltpu.VMEM((PER_W,), jnp.int32),
            pltpu.VMEM((PER_W,), jnp.int32),
            pltpu.VMEM((PER_W,), jnp.float32),
            pltpu.SemaphoreType.DMA,
        ],
    )


def kernel(batch_rel_pos, batch_time_pos, rel_emb, time_emb, W1, b1, W2, b2):
    b, s = batch_rel_pos.shape
    time_emb_p = jnp.zeros((TBL_W, EMBED), jnp.float32).at[:NUM_DT].set(time_emb)
    wa = W1[:, :EMBED]
    wb = W1[:, EMBED:]
    table = _table_call(rel_emb, time_emb_p, wa, wb,
                        b1.reshape(1, EMBED), W2.reshape(1, EMBED),
                        b2.reshape(1, 1))
    rel_flat = batch_rel_pos.reshape(-1).astype(jnp.int32)
    dt_flat = batch_time_pos.reshape(-1).astype(jnp.int32)
    out_flat = _make_gather_kernel()(table.reshape(-1), rel_flat, dt_flat)
    return out_flat.reshape(b, s)
